# baseline (speedup 1.0000x reference)
import functools

import jax
import jax.numpy as jnp
from jax import lax
from jax.experimental import pallas as pl
from jax.experimental.pallas import tpu as pltpu
from jax.experimental.pallas import tpu_sc as plsc

N = 10000
E = 320000
DF = 128
H = 32
NG = 256
K = 60
NCLS = 4
DP = 128

NC, NS = 2, 16
NW = NC * NS
ECH = 64
EPW = 10240
ENCH = EPW // ECH
EPAD = NW * EPW
PCH = 80
NACC = 10112
NPR = NACC // NS
NPW = 320
NPAD = NW * NPW
SLOTS = NG * K
DUMP = SLOTS
SLOTS_PAD = 16000
NB = 5000
BI = 256
NBLK = 40
CJ = 512
GB = 64

_mesh = plsc.VectorSubcoreMesh(core_axis_name="c", subcore_axis_name="s")
_sc_params = pltpu.CompilerParams(use_tc_tiling_on_sc=False)


def _sc_counts(dstw):
    @functools.partial(
        pl.kernel, mesh=_mesh, compiler_params=_sc_params,
        out_type=jax.ShapeDtypeStruct((NC, NACC, 16), jnp.float32),
        scratch_types=[
            pltpu.VMEM((ENCH, ECH), jnp.int32),
            pltpu.VMEM((ECH, 16), jnp.float32),
            pltpu.VMEM((NPR, 16), jnp.float32),
            pltpu.VMEM_SHARED((NACC, 16), jnp.float32),
        ])
    def k(dst_hbm, out_hbm, dst_v, ones_v, zero_v, acc_sh):
        c = lax.axis_index("c")
        s = lax.axis_index("s")
        wid = c * NS + s
        pltpu.sync_copy(dst_hbm.at[wid], dst_v)
        z16 = jnp.zeros((16,), jnp.float32)
        o16 = jnp.full((16,), 1.0, jnp.float32)

        @pl.loop(0, ECH)
        def _(i):
            ones_v[i, pl.ds(0, 16)] = o16

        @pl.loop(0, NPR)
        def _(i):
            zero_v[i, pl.ds(0, 16)] = z16

        pltpu.sync_copy(zero_v, acc_sh.at[pl.ds(s * NPR, NPR)])
        plsc.subcore_barrier()

        @pl.loop(0, ENCH)
        def _(j):
            pltpu.sync_copy(ones_v, acc_sh.at[dst_v.at[j]], add=True)

        plsc.subcore_barrier()
        pltpu.sync_copy(acc_sh.at[pl.ds(s * NPR, NPR)],
                        out_hbm.at[c, pl.ds(s * NPR, NPR)])

    return k(dstw)


def _sc_agg(m, srcw, dstw, w):
    @functools.partial(
        pl.kernel, mesh=_mesh, compiler_params=_sc_params,
        out_type=jax.ShapeDtypeStruct((NC, NACC, w), jnp.float32),
        scratch_types=[
            pltpu.VMEM((ENCH, ECH), jnp.int32),
            pltpu.VMEM((ENCH, ECH), jnp.int32),
            pltpu.VMEM((ECH, w), jnp.float32),
            pltpu.VMEM((ECH, w), jnp.float32),
            pltpu.VMEM((ECH, w), jnp.float32),
            pltpu.VMEM((ECH, w), jnp.float32),
            pltpu.VMEM((NPR, w), jnp.float32),
            pltpu.VMEM_SHARED((NACC, w), jnp.float32),
            pltpu.SemaphoreType.DMA,
            pltpu.SemaphoreType.DMA,
            pltpu.SemaphoreType.DMA,
            pltpu.SemaphoreType.DMA,
        ])
    def k(m_hbm, src_hbm, dst_hbm, out_hbm, src_v, dst_v, r0, r1, r2, r3,
          zero_v, acc_sh, s0, s1, s2, s3):
        c = lax.axis_index("c")
        s = lax.axis_index("s")
        wid = c * NS + s
        pltpu.sync_copy(src_hbm.at[wid], src_v)
        pltpu.sync_copy(dst_hbm.at[wid], dst_v)
        z16 = jnp.zeros((16,), jnp.float32)

        def gath(j, rbuf, sem):
            pltpu.async_copy(m_hbm.at[src_v.at[j]], rbuf, sem)

        def gwait(rbuf, sem):
            pltpu.make_async_copy(m_hbm.at[src_v.at[0]], rbuf, sem).wait()

        gath(0, r0, s0)
        gath(1, r1, s1)
        gath(2, r2, s2)

        @pl.loop(0, NPR)
        def _(i):
            @pl.loop(0, w, step=16)
            def _(jc):
                zero_v[i, pl.ds(jc, 16)] = z16

        pltpu.sync_copy(zero_v, acc_sh.at[pl.ds(s * NPR, NPR)])
        plsc.subcore_barrier()

        @pl.loop(0, ENCH, step=4)
        def _(j):
            gath(j + 3, r3, s3)
            gwait(r0, s0)
            pltpu.sync_copy(r0, acc_sh.at[dst_v.at[j]], add=True)

            @pl.when(j + 4 < ENCH)
            def _():
                gath(j + 4, r0, s0)

            gwait(r1, s1)
            pltpu.sync_copy(r1, acc_sh.at[dst_v.at[j + 1]], add=True)

            @pl.when(j + 5 < ENCH)
            def _():
                gath(j + 5, r1, s1)

            gwait(r2, s2)
            pltpu.sync_copy(r2, acc_sh.at[dst_v.at[j + 2]], add=True)

            @pl.when(j + 6 < ENCH)
            def _():
                gath(j + 6, r2, s2)

            gwait(r3, s3)
            pltpu.sync_copy(r3, acc_sh.at[dst_v.at[j + 3]], add=True)

            @pl.when(j + 7 < ENCH)
            def _():
                gath(j + 7, r3, s3)

        plsc.subcore_barrier()
        pltpu.sync_copy(acc_sh.at[pl.ds(s * NPR, NPR)],
                        out_hbm.at[c, pl.ds(s * NPR, NPR)])

    return k(m, srcw, dstw)


def _sc_pool(featp, slotw):
    @functools.partial(
        pl.kernel, mesh=_mesh, compiler_params=_sc_params,
        out_type=jax.ShapeDtypeStruct((NC, SLOTS_PAD, DP), jnp.float32),
        scratch_types=[
            pltpu.VMEM((NPW // PCH, PCH), jnp.int32),
            pltpu.VMEM((NPW, DP), jnp.float32),
            pltpu.VMEM((200, DP), jnp.float32),
        ])
    def k(f_hbm, sl_hbm, out_hbm, sl_v, f_v, zero_v):
        c = lax.axis_index("c")
        s = lax.axis_index("s")
        wid = c * NS + s
        z16 = jnp.zeros((16,), jnp.float32)

        @pl.loop(0, 200)
        def _(i):
            @pl.loop(0, DP, step=16)
            def _(j):
                zero_v[i, pl.ds(j, 16)] = z16

        @pl.loop(0, SLOTS_PAD // NS // 200)
        def _(t):
            pltpu.sync_copy(
                zero_v,
                out_hbm.at[c, pl.ds(s * (SLOTS_PAD // NS) + t * 200, 200)])

        pltpu.sync_copy(f_hbm.at[pl.ds(wid * NPW, NPW)], f_v)
        pltpu.sync_copy(sl_hbm.at[wid], sl_v)
        plsc.subcore_barrier()

        @pl.loop(0, NPW // PCH)
        def _(t):
            pltpu.sync_copy(f_v.at[pl.ds(t * PCH, PCH)],
                            out_hbm.at[c].at[sl_v.at[t]])

    return k(featp, slotw)


def _tc_prep1(x, W1, counts):
    def body(c_ref, x_ref, w_ref, g_ref, m_ref, d_ref):
        cnt = c_ref[0, :, 0:1] + c_ref[1, :, 0:1]
        dinv = lax.rsqrt(cnt + 1.0)
        g = jnp.dot(x_ref[...], w_ref[...], preferred_element_type=jnp.float32)
        g_ref[...] = g
        m_ref[...] = g * dinv
        d_ref[...] = dinv

    return pl.pallas_call(
        body,
        grid=(N // NB,),
        in_specs=[
            pl.BlockSpec((NC, NB, 16), lambda i: (0, i, 0)),
            pl.BlockSpec((NB, DF), lambda i: (i, 0)),
            pl.BlockSpec((DF, H), lambda i: (0, 0)),
        ],
        out_specs=[
            pl.BlockSpec((NB, H), lambda i: (i, 0)),
            pl.BlockSpec((NB, H), lambda i: (i, 0)),
            pl.BlockSpec((NB, 1), lambda i: (i, 0)),
        ],
        out_shape=[
            jax.ShapeDtypeStruct((N, H), jnp.float32),
            jax.ShapeDtypeStruct((N, H), jnp.float32),
            jax.ShapeDtypeStruct((N, 1), jnp.float32),
        ],
    )(counts, x, W1)


def _tc_layer(acc, g, dinv, b, Wn):
    def body(a_ref, g_ref, d_ref, b_ref, w_ref, h_ref, gn_ref, mn_ref):
        d = d_ref[...]
        g = g_ref[...]
        agg = a_ref[0] + a_ref[1]
        h = jnp.tanh(d * agg + d * d * g + b_ref[...])
        h_ref[...] = h
        gn = jnp.dot(h, w_ref[...], preferred_element_type=jnp.float32)
        gn_ref[...] = gn
        mn_ref[...] = gn * d

    return pl.pallas_call(
        body,
        grid=(N // NB,),
        in_specs=[
            pl.BlockSpec((NC, NB, H), lambda i: (0, i, 0)),
            pl.BlockSpec((NB, H), lambda i: (i, 0)),
            pl.BlockSpec((NB, 1), lambda i: (i, 0)),
            pl.BlockSpec((1, H), lambda i: (0, 0)),
            pl.BlockSpec((H, H), lambda i: (0, 0)),
        ],
        out_specs=[
            pl.BlockSpec((NB, H), lambda i: (i, 0)),
            pl.BlockSpec((NB, H), lambda i: (i, 0)),
            pl.BlockSpec((NB, H), lambda i: (i, 0)),
        ],
        out_shape=[
            jax.ShapeDtypeStruct((N, H), jnp.float32),
            jax.ShapeDtypeStruct((N, H), jnp.float32),
            jax.ShapeDtypeStruct((N, H), jnp.float32),
        ],
    )(acc, g, dinv, b, Wn)


def _tc_layer3(acc, g, dinv, b, W4):
    def body(a_ref, g_ref, d_ref, b_ref, w_ref, h_ref, g4_ref, mn_ref):
        d = d_ref[...]
        g = g_ref[...]
        agg = a_ref[0] + a_ref[1]
        h = jnp.tanh(d * agg + d * d * g + b_ref[...])
        h_ref[...] = h
        g4 = jnp.dot(h, w_ref[...], preferred_element_type=jnp.float32)
        g4_ref[...] = g4
        mn_ref[:, 0:1] = g4 * d
        mn_ref[:, 1:16] = jnp.zeros((NB, 15), jnp.float32)

    return pl.pallas_call(
        body,
        grid=(N // NB,),
        in_specs=[
            pl.BlockSpec((NC, NB, H), lambda i: (0, i, 0)),
            pl.BlockSpec((NB, H), lambda i: (i, 0)),
            pl.BlockSpec((NB, 1), lambda i: (i, 0)),
            pl.BlockSpec((1, H), lambda i: (0, 0)),
            pl.BlockSpec((H, 1), lambda i: (0, 0)),
        ],
        out_specs=[
            pl.BlockSpec((NB, H), lambda i: (i, 0)),
            pl.BlockSpec((NB, 1), lambda i: (i, 0)),
            pl.BlockSpec((NB, 16), lambda i: (i, 0)),
        ],
        out_shape=[
            jax.ShapeDtypeStruct((N, H), jnp.float32),
            jax.ShapeDtypeStruct((N, 1), jnp.float32),
            jax.ShapeDtypeStruct((N, 16), jnp.float32),
        ],
    )(acc, g, dinv, b, W4)


def _tc_post4(acc4, h1, h2, h3, g4w, dinv, b4):
    def body(a_ref, h1_ref, h2_ref, h3_ref, g4_ref, d_ref, b4_ref,
             f_ref, kc_ref):
        d = d_ref[...]
        g4 = g4_ref[...]
        a0 = a_ref[0, :, 0:1] + a_ref[1, :, 0:1]
        h4 = jnp.tanh(d * a0 + d * d * g4 + b4_ref[...])
        f_ref[:, 0:H] = h1_ref[...]
        f_ref[:, H:2 * H] = h2_ref[...]
        f_ref[:, 2 * H:3 * H] = h3_ref[...]
        f_ref[:, 3 * H:3 * H + 1] = h4
        f_ref[:, 3 * H + 1:DP] = jnp.zeros((NB, DP - 3 * H - 1), jnp.float32)
        kc_ref[...] = h4

    return pl.pallas_call(
        body,
        grid=(N // NB,),
        in_specs=[
            pl.BlockSpec((NC, NB, 16), lambda i: (0, i, 0)),
            pl.BlockSpec((NB, H), lambda i: (i, 0)),
            pl.BlockSpec((NB, H), lambda i: (i, 0)),
            pl.BlockSpec((NB, H), lambda i: (i, 0)),
            pl.BlockSpec((NB, 1), lambda i: (i, 0)),
            pl.BlockSpec((NB, 1), lambda i: (i, 0)),
            pl.BlockSpec((1, 1), lambda i: (0, 0)),
        ],
        out_specs=[
            pl.BlockSpec((NB, DP), lambda i: (i, 0)),
            pl.BlockSpec((NB, 1), lambda i: (i, 0)),
        ],
        out_shape=[
            jax.ShapeDtypeStruct((NPAD, DP), jnp.float32),
            jax.ShapeDtypeStruct((N, 1), jnp.float32),
        ],
    )(acc4, h1, h2, h3, g4w, dinv, b4)


def _tc_rank(key_row, batch_row, key_col, batch_col):
    def body(kr_ref, br_ref, kc_ref, bc_ref, s_ref):
        i = pl.program_id(0)
        br_all = br_ref[...]
        b0 = bc_ref[0, 0]
        b1 = bc_ref[BI - 1, 0]
        lo = jnp.sum((br_all < b0).astype(jnp.int32))
        hi = jnp.sum((br_all <= b1).astype(jnp.int32))
        ki = kc_ref[...]
        bi = bc_ref[...]
        gi = i * BI + lax.broadcasted_iota(jnp.int32, (BI, 1), 0)

        def jbody(cc, cnt):
            off = pl.multiple_of(cc * CJ, CJ)
            kj = kr_ref[:, pl.ds(off, CJ)]
            bj = br_ref[:, pl.ds(off, CJ)]
            jj = off + lax.broadcasted_iota(jnp.int32, (1, CJ), 1)
            beat = (bj == bi) & ((kj > ki) | ((kj == ki) & (jj < gi)))
            return cnt + jnp.sum(beat.astype(jnp.int32), axis=1, keepdims=True)

        cnt = lax.fori_loop(lo // CJ, (hi + CJ - 1) // CJ, jbody,
                            jnp.zeros((BI, 1), jnp.int32))
        s_ref[...] = jnp.where((cnt < K) & (gi < N), bi * K + cnt, DUMP)

    return pl.pallas_call(
        body,
        grid=(NBLK,),
        in_specs=[
            pl.BlockSpec((1, NPAD), lambda i: (0, 0)),
            pl.BlockSpec((1, NPAD), lambda i: (0, 0)),
            pl.BlockSpec((BI, 1), lambda i: (i, 0)),
            pl.BlockSpec((BI, 1), lambda i: (i, 0)),
        ],
        out_specs=pl.BlockSpec((BI, 1), lambda i: (i, 0)),
        out_shape=jax.ShapeDtypeStruct((NPAD, 1), jnp.int32),
    )(key_row, batch_row, key_col, batch_col)


def _tc_head(pool, cw1r, cb1, w2r, cb2, lw1p, lb1, lw2, lb2):
    def body(p_ref, c1w, c1b, c2w, c2b, l1w, l1b, l2w, l2b, o_ref):
        P = (p_ref[0] + p_ref[1])[:, 0:3 * H + 1]
        c1 = jnp.dot(P, c1w[...], preferred_element_type=jnp.float32)
        c1 = jnp.maximum(c1 + c1b[...], 0.0)
        y = jnp.max(c1.reshape(GB, K // 2, 2, 16), axis=2)
        z = jnp.zeros((GB * 26, 32), jnp.float32) + c2b[...]
        for dt in range(5):
            ydt = y[:, dt:dt + 26, :].reshape(GB * 26, 16)
            z = z + jnp.dot(ydt, c2w[pl.ds(dt * 16, 16), :],
                            preferred_element_type=jnp.float32)
        z3 = jnp.maximum(z, 0.0).reshape(GB, 26, 32)
        q = l1b[...] + jnp.zeros((GB, DF), jnp.float32)
        for t in range(26):
            q = q + jnp.dot(z3[:, t, :], l1w[pl.ds(t * 32, 32), :],
                            preferred_element_type=jnp.float32)
        q = jnp.maximum(q, 0.0)
        o = jnp.dot(q, l2w[...], preferred_element_type=jnp.float32) + l2b[...]
        mo = jnp.max(o, axis=1, keepdims=True)
        e = jnp.exp(o - mo)
        o_ref[...] = e / jnp.sum(e, axis=1, keepdims=True)

    return pl.pallas_call(
        body,
        grid=(NG // GB,),
        in_specs=[
            pl.BlockSpec((NC, GB * K, DP), lambda i: (0, i, 0)),
            pl.BlockSpec((3 * H + 1, 16), lambda i: (0, 0)),
            pl.BlockSpec((1, 16), lambda i: (0, 0)),
            pl.BlockSpec((80, 32), lambda i: (0, 0)),
            pl.BlockSpec((1, 32), lambda i: (0, 0)),
            pl.BlockSpec((832, DF), lambda i: (0, 0)),
            pl.BlockSpec((1, DF), lambda i: (0, 0)),
            pl.BlockSpec((DF, NCLS), lambda i: (0, 0)),
            pl.BlockSpec((1, NCLS), lambda i: (0, 0)),
        ],
        out_specs=pl.BlockSpec((GB, NCLS), lambda i: (i, 0)),
        out_shape=jax.ShapeDtypeStruct((NG, NCLS), jnp.float32),
    )(pool, cw1r, cb1, w2r, cb2, lw1p, lb1, lw2, lb2)


def kernel(x, edge_index, batch, W1, b1, W2, b2, W3, b3, W4, b4,
           cw1, cb1, cw2, cb2, lw1, lb1, lw2, lb2):
    src = edge_index[0].astype(jnp.int32)
    dst = edge_index[1].astype(jnp.int32)
    srcw = jnp.concatenate(
        [src, jnp.zeros((EPAD - E,), jnp.int32)]).reshape(NW, ENCH, ECH)
    dstw = jnp.concatenate(
        [dst, jnp.full((EPAD - E,), NACC - 1, jnp.int32)]).reshape(NW, ENCH, ECH)

    counts = _sc_counts(dstw)

    g1, m1, dinv = _tc_prep1(x, W1, counts)

    acc1 = _sc_agg(m1, srcw, dstw, H)
    h1, g2, m2 = _tc_layer(acc1, g1, dinv, b1.reshape(1, H), W2)
    acc2 = _sc_agg(m2, srcw, dstw, H)
    h2, g3, m3 = _tc_layer(acc2, g2, dinv, b2.reshape(1, H), W3)
    acc3 = _sc_agg(m3, srcw, dstw, H)
    h3, g4w, m4n = _tc_layer3(acc3, g3, dinv, b3.reshape(1, H), W4)
    acc4 = _sc_agg(m4n, srcw, dstw, 16)
    featp, key_col = _tc_post4(acc4, h1, h2, h3, g4w, dinv, b4.reshape(1, 1))

    key_row = jnp.pad(key_col.reshape(1, N), ((0, 0), (0, NPAD - N)))
    batch32 = batch.astype(jnp.int32)
    batch_pad = jnp.concatenate([batch32, jnp.full((NPAD - N,), NG, jnp.int32)])
    key_colp = jnp.pad(key_col, ((0, NPAD - N), (0, 0)))
    batch_row = batch_pad.reshape(1, NPAD)
    batch_col = batch_pad.reshape(NPAD, 1)
    slot = _tc_rank(key_row, batch_row, key_colp, batch_col)

    slotw = slot.reshape(NW, NPW // PCH, PCH)
    pool = _sc_pool(featp, slotw)

    cw1r = cw1[:, 0, :].T
    w2r = cw2.transpose(2, 1, 0).reshape(80, 32)
    lw1p = lw1.reshape(32, 26, DF).transpose(1, 0, 2).reshape(832, DF)
    return _tc_head(pool, cw1r, cb1.reshape(1, 16), w2r, cb2.reshape(1, 32),
                    lw1p, lb1.reshape(1, DF), lw2, lb2.reshape(1, NCLS))


# speedup vs baseline: 1.0045x; 1.0045x over previous
import functools

import jax
import jax.numpy as jnp
from jax import lax
from jax.experimental import pallas as pl
from jax.experimental.pallas import tpu as pltpu
from jax.experimental.pallas import tpu_sc as plsc

N = 10000
E = 320000
DF = 128
H = 32
NG = 256
K = 60
NCLS = 4
DP = 128

NC, NS = 2, 16
NW = NC * NS
ECH = 80
EPW = 10240
ENCH = EPW // ECH
EPAD = NW * EPW
PCH = 80
NACC = 10112
NPR = NACC // NS
NPW = 320
NPAD = NW * NPW
SLOTS = NG * K
DUMP = SLOTS
SLOTS_PAD = 16000
NB = 5000
BI = 256
NBLK = 40
CJ = 512
GB = 64

_mesh = plsc.VectorSubcoreMesh(core_axis_name="c", subcore_axis_name="s")
_sc_params = pltpu.CompilerParams(use_tc_tiling_on_sc=False)


def _sc_counts(dstw):
    @functools.partial(
        pl.kernel, mesh=_mesh, compiler_params=_sc_params,
        out_type=jax.ShapeDtypeStruct((NC, NACC, 16), jnp.float32),
        scratch_types=[
            pltpu.VMEM((ENCH, ECH), jnp.int32),
            pltpu.VMEM((ECH, 16), jnp.float32),
            pltpu.VMEM((NPR, 16), jnp.float32),
            pltpu.VMEM_SHARED((NACC, 16), jnp.float32),
        ])
    def k(dst_hbm, out_hbm, dst_v, ones_v, zero_v, acc_sh):
        c = lax.axis_index("c")
        s = lax.axis_index("s")
        wid = c * NS + s
        pltpu.sync_copy(dst_hbm.at[wid], dst_v)
        z16 = jnp.zeros((16,), jnp.float32)
        o16 = jnp.full((16,), 1.0, jnp.float32)

        @pl.loop(0, ECH)
        def _(i):
            ones_v[i, pl.ds(0, 16)] = o16

        @pl.loop(0, NPR)
        def _(i):
            zero_v[i, pl.ds(0, 16)] = z16

        pltpu.sync_copy(zero_v, acc_sh.at[pl.ds(s * NPR, NPR)])
        plsc.subcore_barrier()

        @pl.loop(0, ENCH)
        def _(j):
            pltpu.sync_copy(ones_v, acc_sh.at[dst_v.at[j]], add=True)

        plsc.subcore_barrier()
        pltpu.sync_copy(acc_sh.at[pl.ds(s * NPR, NPR)],
                        out_hbm.at[c, pl.ds(s * NPR, NPR)])

    return k(dstw)


def _sc_agg(m, srcw, dstw, w):
    @functools.partial(
        pl.kernel, mesh=_mesh, compiler_params=_sc_params,
        out_type=jax.ShapeDtypeStruct((NC, NACC, w), jnp.float32),
        scratch_types=[
            pltpu.VMEM((ENCH, ECH), jnp.int32),
            pltpu.VMEM((ENCH, ECH), jnp.int32),
            pltpu.VMEM((ECH, w), jnp.float32),
            pltpu.VMEM((ECH, w), jnp.float32),
            pltpu.VMEM((ECH, w), jnp.float32),
            pltpu.VMEM((ECH, w), jnp.float32),
            pltpu.VMEM((NPR, w), jnp.float32),
            pltpu.VMEM_SHARED((NACC, w), jnp.float32),
            pltpu.SemaphoreType.DMA,
            pltpu.SemaphoreType.DMA,
            pltpu.SemaphoreType.DMA,
            pltpu.SemaphoreType.DMA,
        ])
    def k(m_hbm, src_hbm, dst_hbm, out_hbm, src_v, dst_v, r0, r1, r2, r3,
          zero_v, acc_sh, s0, s1, s2, s3):
        c = lax.axis_index("c")
        s = lax.axis_index("s")
        wid = c * NS + s
        pltpu.sync_copy(src_hbm.at[wid], src_v)
        pltpu.sync_copy(dst_hbm.at[wid], dst_v)
        z16 = jnp.zeros((16,), jnp.float32)

        def gath(j, rbuf, sem):
            pltpu.async_copy(m_hbm.at[src_v.at[j]], rbuf, sem)

        def gwait(rbuf, sem):
            pltpu.make_async_copy(m_hbm.at[src_v.at[0]], rbuf, sem).wait()

        gath(0, r0, s0)
        gath(1, r1, s1)
        gath(2, r2, s2)

        @pl.loop(0, NPR)
        def _(i):
            @pl.loop(0, w, step=16)
            def _(jc):
                zero_v[i, pl.ds(jc, 16)] = z16

        pltpu.sync_copy(zero_v, acc_sh.at[pl.ds(s * NPR, NPR)])
        plsc.subcore_barrier()

        @pl.loop(0, ENCH, step=4)
        def _(j):
            gath(j + 3, r3, s3)
            gwait(r0, s0)
            pltpu.sync_copy(r0, acc_sh.at[dst_v.at[j]], add=True)

            @pl.when(j + 4 < ENCH)
            def _():
                gath(j + 4, r0, s0)

            gwait(r1, s1)
            pltpu.sync_copy(r1, acc_sh.at[dst_v.at[j + 1]], add=True)

            @pl.when(j + 5 < ENCH)
            def _():
                gath(j + 5, r1, s1)

            gwait(r2, s2)
            pltpu.sync_copy(r2, acc_sh.at[dst_v.at[j + 2]], add=True)

            @pl.when(j + 6 < ENCH)
            def _():
                gath(j + 6, r2, s2)

            gwait(r3, s3)
            pltpu.sync_copy(r3, acc_sh.at[dst_v.at[j + 3]], add=True)

            @pl.when(j + 7 < ENCH)
            def _():
                gath(j + 7, r3, s3)

        plsc.subcore_barrier()
        pltpu.sync_copy(acc_sh.at[pl.ds(s * NPR, NPR)],
                        out_hbm.at[c, pl.ds(s * NPR, NPR)])

    return k(m, srcw, dstw)


def _sc_pool(featp, slotw):
    @functools.partial(
        pl.kernel, mesh=_mesh, compiler_params=_sc_params,
        out_type=jax.ShapeDtypeStruct((NC, SLOTS_PAD, DP), jnp.float32),
        scratch_types=[
            pltpu.VMEM((NPW // PCH, PCH), jnp.int32),
            pltpu.VMEM((NPW, DP), jnp.float32),
            pltpu.VMEM((200, DP), jnp.float32),
        ])
    def k(f_hbm, sl_hbm, out_hbm, sl_v, f_v, zero_v):
        c = lax.axis_index("c")
        s = lax.axis_index("s")
        wid = c * NS + s
        z16 = jnp.zeros((16,), jnp.float32)

        @pl.loop(0, 200)
        def _(i):
            @pl.loop(0, DP, step=16)
            def _(j):
                zero_v[i, pl.ds(j, 16)] = z16

        @pl.loop(0, SLOTS_PAD // NS // 200)
        def _(t):
            pltpu.sync_copy(
                zero_v,
                out_hbm.at[c, pl.ds(s * (SLOTS_PAD // NS) + t * 200, 200)])

        pltpu.sync_copy(f_hbm.at[pl.ds(wid * NPW, NPW)], f_v)
        pltpu.sync_copy(sl_hbm.at[wid], sl_v)
        plsc.subcore_barrier()

        @pl.loop(0, NPW // PCH)
        def _(t):
            pltpu.sync_copy(f_v.at[pl.ds(t * PCH, PCH)],
                            out_hbm.at[c].at[sl_v.at[t]])

    return k(featp, slotw)


def _tc_prep1(x, W1, counts):
    def body(c_ref, x_ref, w_ref, g_ref, m_ref, d_ref):
        cnt = c_ref[0, :, 0:1] + c_ref[1, :, 0:1]
        dinv = lax.rsqrt(cnt + 1.0)
        g = jnp.dot(x_ref[...], w_ref[...], preferred_element_type=jnp.float32)
        g_ref[...] = g
        m_ref[...] = g * dinv
        d_ref[...] = dinv

    return pl.pallas_call(
        body,
        grid=(N // NB,),
        in_specs=[
            pl.BlockSpec((NC, NB, 16), lambda i: (0, i, 0)),
            pl.BlockSpec((NB, DF), lambda i: (i, 0)),
            pl.BlockSpec((DF, H), lambda i: (0, 0)),
        ],
        out_specs=[
            pl.BlockSpec((NB, H), lambda i: (i, 0)),
            pl.BlockSpec((NB, H), lambda i: (i, 0)),
            pl.BlockSpec((NB, 1), lambda i: (i, 0)),
        ],
        out_shape=[
            jax.ShapeDtypeStruct((N, H), jnp.float32),
            jax.ShapeDtypeStruct((N, H), jnp.float32),
            jax.ShapeDtypeStruct((N, 1), jnp.float32),
        ],
    )(counts, x, W1)


def _tc_layer(acc, g, dinv, b, Wn):
    def body(a_ref, g_ref, d_ref, b_ref, w_ref, h_ref, gn_ref, mn_ref):
        d = d_ref[...]
        g = g_ref[...]
        agg = a_ref[0] + a_ref[1]
        h = jnp.tanh(d * agg + d * d * g + b_ref[...])
        h_ref[...] = h
        gn = jnp.dot(h, w_ref[...], preferred_element_type=jnp.float32)
        gn_ref[...] = gn
        mn_ref[...] = gn * d

    return pl.pallas_call(
        body,
        grid=(N // NB,),
        in_specs=[
            pl.BlockSpec((NC, NB, H), lambda i: (0, i, 0)),
            pl.BlockSpec((NB, H), lambda i: (i, 0)),
            pl.BlockSpec((NB, 1), lambda i: (i, 0)),
            pl.BlockSpec((1, H), lambda i: (0, 0)),
            pl.BlockSpec((H, H), lambda i: (0, 0)),
        ],
        out_specs=[
            pl.BlockSpec((NB, H), lambda i: (i, 0)),
            pl.BlockSpec((NB, H), lambda i: (i, 0)),
            pl.BlockSpec((NB, H), lambda i: (i, 0)),
        ],
        out_shape=[
            jax.ShapeDtypeStruct((N, H), jnp.float32),
            jax.ShapeDtypeStruct((N, H), jnp.float32),
            jax.ShapeDtypeStruct((N, H), jnp.float32),
        ],
    )(acc, g, dinv, b, Wn)


def _tc_layer3(acc, g, dinv, b, W4):
    def body(a_ref, g_ref, d_ref, b_ref, w_ref, h_ref, g4_ref, mn_ref):
        d = d_ref[...]
        g = g_ref[...]
        agg = a_ref[0] + a_ref[1]
        h = jnp.tanh(d * agg + d * d * g + b_ref[...])
        h_ref[...] = h
        g4 = jnp.dot(h, w_ref[...], preferred_element_type=jnp.float32)
        g4_ref[...] = g4
        mn_ref[:, 0:1] = g4 * d
        mn_ref[:, 1:16] = jnp.zeros((NB, 15), jnp.float32)

    return pl.pallas_call(
        body,
        grid=(N // NB,),
        in_specs=[
            pl.BlockSpec((NC, NB, H), lambda i: (0, i, 0)),
            pl.BlockSpec((NB, H), lambda i: (i, 0)),
            pl.BlockSpec((NB, 1), lambda i: (i, 0)),
            pl.BlockSpec((1, H), lambda i: (0, 0)),
            pl.BlockSpec((H, 1), lambda i: (0, 0)),
        ],
        out_specs=[
            pl.BlockSpec((NB, H), lambda i: (i, 0)),
            pl.BlockSpec((NB, 1), lambda i: (i, 0)),
            pl.BlockSpec((NB, 16), lambda i: (i, 0)),
        ],
        out_shape=[
            jax.ShapeDtypeStruct((N, H), jnp.float32),
            jax.ShapeDtypeStruct((N, 1), jnp.float32),
            jax.ShapeDtypeStruct((N, 16), jnp.float32),
        ],
    )(acc, g, dinv, b, W4)


def _tc_post4(acc4, h1, h2, h3, g4w, dinv, b4):
    def body(a_ref, h1_ref, h2_ref, h3_ref, g4_ref, d_ref, b4_ref,
             f_ref, kc_ref):
        d = d_ref[...]
        g4 = g4_ref[...]
        a0 = a_ref[0, :, 0:1] + a_ref[1, :, 0:1]
        h4 = jnp.tanh(d * a0 + d * d * g4 + b4_ref[...])
        f_ref[:, 0:H] = h1_ref[...]
        f_ref[:, H:2 * H] = h2_ref[...]
        f_ref[:, 2 * H:3 * H] = h3_ref[...]
        f_ref[:, 3 * H:3 * H + 1] = h4
        f_ref[:, 3 * H + 1:DP] = jnp.zeros((NB, DP - 3 * H - 1), jnp.float32)
        kc_ref[...] = h4

    return pl.pallas_call(
        body,
        grid=(N // NB,),
        in_specs=[
            pl.BlockSpec((NC, NB, 16), lambda i: (0, i, 0)),
            pl.BlockSpec((NB, H), lambda i: (i, 0)),
            pl.BlockSpec((NB, H), lambda i: (i, 0)),
            pl.BlockSpec((NB, H), lambda i: (i, 0)),
            pl.BlockSpec((NB, 1), lambda i: (i, 0)),
            pl.BlockSpec((NB, 1), lambda i: (i, 0)),
            pl.BlockSpec((1, 1), lambda i: (0, 0)),
        ],
        out_specs=[
            pl.BlockSpec((NB, DP), lambda i: (i, 0)),
            pl.BlockSpec((NB, 1), lambda i: (i, 0)),
        ],
        out_shape=[
            jax.ShapeDtypeStruct((NPAD, DP), jnp.float32),
            jax.ShapeDtypeStruct((N, 1), jnp.float32),
        ],
    )(acc4, h1, h2, h3, g4w, dinv, b4)


def _tc_rank(key_row, batch_row, key_col, batch_col):
    def body(kr_ref, br_ref, kc_ref, bc_ref, s_ref):
        i = pl.program_id(0)
        br_all = br_ref[...]
        b0 = bc_ref[0, 0]
        b1 = bc_ref[BI - 1, 0]
        lo = jnp.sum((br_all < b0).astype(jnp.int32))
        hi = jnp.sum((br_all <= b1).astype(jnp.int32))
        ki = kc_ref[...]
        bi = bc_ref[...]
        gi = i * BI + lax.broadcasted_iota(jnp.int32, (BI, 1), 0)

        def jbody(cc, cnt):
            off = pl.multiple_of(cc * CJ, CJ)
            kj = kr_ref[:, pl.ds(off, CJ)]
            bj = br_ref[:, pl.ds(off, CJ)]
            jj = off + lax.broadcasted_iota(jnp.int32, (1, CJ), 1)
            beat = (bj == bi) & ((kj > ki) | ((kj == ki) & (jj < gi)))
            return cnt + jnp.sum(beat.astype(jnp.int32), axis=1, keepdims=True)

        cnt = lax.fori_loop(lo // CJ, (hi + CJ - 1) // CJ, jbody,
                            jnp.zeros((BI, 1), jnp.int32))
        s_ref[...] = jnp.where((cnt < K) & (gi < N), bi * K + cnt, DUMP)

    return pl.pallas_call(
        body,
        grid=(NBLK,),
        in_specs=[
            pl.BlockSpec((1, NPAD), lambda i: (0, 0)),
            pl.BlockSpec((1, NPAD), lambda i: (0, 0)),
            pl.BlockSpec((BI, 1), lambda i: (i, 0)),
            pl.BlockSpec((BI, 1), lambda i: (i, 0)),
        ],
        out_specs=pl.BlockSpec((BI, 1), lambda i: (i, 0)),
        out_shape=jax.ShapeDtypeStruct((NPAD, 1), jnp.int32),
    )(key_row, batch_row, key_col, batch_col)


def _tc_head(pool, cw1r, cb1, w2r, cb2, lw1p, lb1, lw2, lb2):
    def body(p_ref, c1w, c1b, c2w, c2b, l1w, l1b, l2w, l2b, o_ref):
        P = (p_ref[0] + p_ref[1])[:, 0:3 * H + 1]
        c1 = jnp.dot(P, c1w[...], preferred_element_type=jnp.float32)
        c1 = jnp.maximum(c1 + c1b[...], 0.0)
        y = jnp.max(c1.reshape(GB, K // 2, 2, 16), axis=2)
        z = jnp.zeros((GB * 26, 32), jnp.float32) + c2b[...]
        for dt in range(5):
            ydt = y[:, dt:dt + 26, :].reshape(GB * 26, 16)
            z = z + jnp.dot(ydt, c2w[pl.ds(dt * 16, 16), :],
                            preferred_element_type=jnp.float32)
        z3 = jnp.maximum(z, 0.0).reshape(GB, 26, 32)
        q = l1b[...] + jnp.zeros((GB, DF), jnp.float32)
        for t in range(26):
            q = q + jnp.dot(z3[:, t, :], l1w[pl.ds(t * 32, 32), :],
                            preferred_element_type=jnp.float32)
        q = jnp.maximum(q, 0.0)
        o = jnp.dot(q, l2w[...], preferred_element_type=jnp.float32) + l2b[...]
        mo = jnp.max(o, axis=1, keepdims=True)
        e = jnp.exp(o - mo)
        o_ref[...] = e / jnp.sum(e, axis=1, keepdims=True)

    return pl.pallas_call(
        body,
        grid=(NG // GB,),
        in_specs=[
            pl.BlockSpec((NC, GB * K, DP), lambda i: (0, i, 0)),
            pl.BlockSpec((3 * H + 1, 16), lambda i: (0, 0)),
            pl.BlockSpec((1, 16), lambda i: (0, 0)),
            pl.BlockSpec((80, 32), lambda i: (0, 0)),
            pl.BlockSpec((1, 32), lambda i: (0, 0)),
            pl.BlockSpec((832, DF), lambda i: (0, 0)),
            pl.BlockSpec((1, DF), lambda i: (0, 0)),
            pl.BlockSpec((DF, NCLS), lambda i: (0, 0)),
            pl.BlockSpec((1, NCLS), lambda i: (0, 0)),
        ],
        out_specs=pl.BlockSpec((GB, NCLS), lambda i: (i, 0)),
        out_shape=jax.ShapeDtypeStruct((NG, NCLS), jnp.float32),
    )(pool, cw1r, cb1, w2r, cb2, lw1p, lb1, lw2, lb2)


def kernel(x, edge_index, batch, W1, b1, W2, b2, W3, b3, W4, b4,
           cw1, cb1, cw2, cb2, lw1, lb1, lw2, lb2):
    src = edge_index[0].astype(jnp.int32)
    dst = edge_index[1].astype(jnp.int32)
    srcw = jnp.concatenate(
        [src, jnp.zeros((EPAD - E,), jnp.int32)]).reshape(NW, ENCH, ECH)
    dstw = jnp.concatenate(
        [dst, jnp.full((EPAD - E,), NACC - 1, jnp.int32)]).reshape(NW, ENCH, ECH)

    counts = _sc_counts(dstw)

    g1, m1, dinv = _tc_prep1(x, W1, counts)

    acc1 = _sc_agg(m1, srcw, dstw, H)
    h1, g2, m2 = _tc_layer(acc1, g1, dinv, b1.reshape(1, H), W2)
    acc2 = _sc_agg(m2, srcw, dstw, H)
    h2, g3, m3 = _tc_layer(acc2, g2, dinv, b2.reshape(1, H), W3)
    acc3 = _sc_agg(m3, srcw, dstw, H)
    h3, g4w, m4n = _tc_layer3(acc3, g3, dinv, b3.reshape(1, H), W4)
    acc4 = _sc_agg(m4n, srcw, dstw, 16)
    featp, key_col = _tc_post4(acc4, h1, h2, h3, g4w, dinv, b4.reshape(1, 1))

    key_row = jnp.pad(key_col.reshape(1, N), ((0, 0), (0, NPAD - N)))
    batch32 = batch.astype(jnp.int32)
    batch_pad = jnp.concatenate([batch32, jnp.full((NPAD - N,), NG, jnp.int32)])
    key_colp = jnp.pad(key_col, ((0, NPAD - N), (0, 0)))
    batch_row = batch_pad.reshape(1, NPAD)
    batch_col = batch_pad.reshape(NPAD, 1)
    slot = _tc_rank(key_row, batch_row, key_colp, batch_col)

    slotw = slot.reshape(NW, NPW // PCH, PCH)
    pool = _sc_pool(featp, slotw)

    cw1r = cw1[:, 0, :].T
    w2r = cw2.transpose(2, 1, 0).reshape(80, 32)
    lw1p = lw1.reshape(32, 26, DF).transpose(1, 0, 2).reshape(832, DF)
    return _tc_head(pool, cw1r, cb1.reshape(1, 16), w2r, cb2.reshape(1, 32),
                    lw1p, lb1.reshape(1, DF), lw2, lb2.reshape(1, NCLS))


# speedup vs baseline: 1.7686x; 1.7606x over previous
import functools

import jax
import jax.numpy as jnp
from jax import lax
from jax.experimental import pallas as pl
from jax.experimental.pallas import tpu as pltpu
from jax.experimental.pallas import tpu_sc as plsc

N = 10000
E = 320000
DF = 128
H = 32
NG = 256
K = 60
NCLS = 4
DP = 128

NC, NS = 2, 16
NW = NC * NS
ECH = 80
EPW = 10240
ENCH = EPW // ECH
EPAD = NW * EPW
PCH = 80
NACC = 10112
NPR = NACC // NS
NPW = 320
NPAD = NW * NPW
SLOTS = NG * K
DUMP = SLOTS
SLOTS_PAD = 16000
NB = 5000
BI = 256
NBLK = 40
CJ = 512
GB = 64

_mesh = plsc.VectorSubcoreMesh(core_axis_name="c", subcore_axis_name="s")
_sc_params = pltpu.CompilerParams(use_tc_tiling_on_sc=False)


def _sc_counts(dstw):
    @functools.partial(
        pl.kernel, mesh=_mesh, compiler_params=_sc_params,
        out_type=jax.ShapeDtypeStruct((NC, NACC, 16), jnp.float32),
        scratch_types=[
            pltpu.VMEM((ENCH, ECH), jnp.int32),
            pltpu.VMEM((ECH, 16), jnp.float32),
            pltpu.VMEM((NPR, 16), jnp.float32),
            pltpu.VMEM_SHARED((NACC, 16), jnp.float32),
        ])
    def k(dst_hbm, out_hbm, dst_v, ones_v, zero_v, acc_sh):
        c = lax.axis_index("c")
        s = lax.axis_index("s")
        wid = c * NS + s
        pltpu.sync_copy(dst_hbm.at[wid], dst_v)
        z16 = jnp.zeros((16,), jnp.float32)
        o16 = jnp.full((16,), 1.0, jnp.float32)

        @pl.loop(0, ECH)
        def _(i):
            ones_v[i, pl.ds(0, 16)] = o16

        @pl.loop(0, NPR)
        def _(i):
            zero_v[i, pl.ds(0, 16)] = z16

        pltpu.sync_copy(zero_v, acc_sh.at[pl.ds(s * NPR, NPR)])
        plsc.subcore_barrier()

        @pl.loop(0, ENCH)
        def _(j):
            pltpu.sync_copy(ones_v, acc_sh.at[dst_v.at[j]], add=True)

        plsc.subcore_barrier()
        pltpu.sync_copy(acc_sh.at[pl.ds(s * NPR, NPR)],
                        out_hbm.at[c, pl.ds(s * NPR, NPR)])

    return k(dstw)


def _sc_agg(m, srcw, dstw, w):
    @functools.partial(
        pl.kernel, mesh=_mesh, compiler_params=_sc_params,
        out_type=jax.ShapeDtypeStruct((NC, NACC, w), jnp.float32),
        scratch_types=[
            pltpu.VMEM((ENCH, ECH), jnp.int32),
            pltpu.VMEM((ENCH, ECH), jnp.int32),
            pltpu.VMEM((ECH, w), jnp.float32),
            pltpu.VMEM((ECH, w), jnp.float32),
            pltpu.VMEM((ECH, w), jnp.float32),
            pltpu.VMEM((ECH, w), jnp.float32),
            pltpu.VMEM((NPR, w), jnp.float32),
            pltpu.VMEM_SHARED((NACC, w), jnp.float32),
            pltpu.VMEM_SHARED((N, w), jnp.float32),
            pltpu.SemaphoreType.DMA,
            pltpu.SemaphoreType.DMA,
            pltpu.SemaphoreType.DMA,
            pltpu.SemaphoreType.DMA,
        ])
    def k(m_hbm, src_hbm, dst_hbm, out_hbm, src_v, dst_v, r0, r1, r2, r3,
          zero_v, acc_sh, m_sh, s0, s1, s2, s3):
        c = lax.axis_index("c")
        s = lax.axis_index("s")
        wid = c * NS + s
        pltpu.sync_copy(src_hbm.at[wid], src_v)
        pltpu.sync_copy(dst_hbm.at[wid], dst_v)
        pltpu.sync_copy(m_hbm.at[pl.ds(s * (N // NS), N // NS)],
                        m_sh.at[pl.ds(s * (N // NS), N // NS)])
        z16 = jnp.zeros((16,), jnp.float32)

        def gath(j, rbuf, sem):
            pltpu.async_copy(m_sh.at[src_v.at[j]], rbuf, sem)

        def gwait(rbuf, sem):
            pltpu.make_async_copy(m_sh.at[src_v.at[0]], rbuf, sem).wait()

        @pl.loop(0, NPR)
        def _(i):
            @pl.loop(0, w, step=16)
            def _(jc):
                zero_v[i, pl.ds(jc, 16)] = z16

        pltpu.sync_copy(zero_v, acc_sh.at[pl.ds(s * NPR, NPR)])
        plsc.subcore_barrier()
        gath(0, r0, s0)
        gath(1, r1, s1)
        gath(2, r2, s2)

        @pl.loop(0, ENCH, step=4)
        def _(j):
            gath(j + 3, r3, s3)
            gwait(r0, s0)
            pltpu.sync_copy(r0, acc_sh.at[dst_v.at[j]], add=True)

            @pl.when(j + 4 < ENCH)
            def _():
                gath(j + 4, r0, s0)

            gwait(r1, s1)
            pltpu.sync_copy(r1, acc_sh.at[dst_v.at[j + 1]], add=True)

            @pl.when(j + 5 < ENCH)
            def _():
                gath(j + 5, r1, s1)

            gwait(r2, s2)
            pltpu.sync_copy(r2, acc_sh.at[dst_v.at[j + 2]], add=True)

            @pl.when(j + 6 < ENCH)
            def _():
                gath(j + 6, r2, s2)

            gwait(r3, s3)
            pltpu.sync_copy(r3, acc_sh.at[dst_v.at[j + 3]], add=True)

            @pl.when(j + 7 < ENCH)
            def _():
                gath(j + 7, r3, s3)

        plsc.subcore_barrier()
        pltpu.sync_copy(acc_sh.at[pl.ds(s * NPR, NPR)],
                        out_hbm.at[c, pl.ds(s * NPR, NPR)])

    return k(m, srcw, dstw)


def _sc_pool(featp, slotw):
    @functools.partial(
        pl.kernel, mesh=_mesh, compiler_params=_sc_params,
        out_type=jax.ShapeDtypeStruct((NC, SLOTS_PAD, DP), jnp.float32),
        scratch_types=[
            pltpu.VMEM((NPW // PCH, PCH), jnp.int32),
            pltpu.VMEM((NPW, DP), jnp.float32),
            pltpu.VMEM((200, DP), jnp.float32),
        ])
    def k(f_hbm, sl_hbm, out_hbm, sl_v, f_v, zero_v):
        c = lax.axis_index("c")
        s = lax.axis_index("s")
        wid = c * NS + s
        z16 = jnp.zeros((16,), jnp.float32)

        @pl.loop(0, 200)
        def _(i):
            @pl.loop(0, DP, step=16)
            def _(j):
                zero_v[i, pl.ds(j, 16)] = z16

        @pl.loop(0, SLOTS_PAD // NS // 200)
        def _(t):
            pltpu.sync_copy(
                zero_v,
                out_hbm.at[c, pl.ds(s * (SLOTS_PAD // NS) + t * 200, 200)])

        pltpu.sync_copy(f_hbm.at[pl.ds(wid * NPW, NPW)], f_v)
        pltpu.sync_copy(sl_hbm.at[wid], sl_v)
        plsc.subcore_barrier()

        @pl.loop(0, NPW // PCH)
        def _(t):
            pltpu.sync_copy(f_v.at[pl.ds(t * PCH, PCH)],
                            out_hbm.at[c].at[sl_v.at[t]])

    return k(featp, slotw)


def _tc_prep1(x, W1, counts):
    def body(c_ref, x_ref, w_ref, g_ref, m_ref, d_ref):
        cnt = c_ref[0, :, 0:1] + c_ref[1, :, 0:1]
        dinv = lax.rsqrt(cnt + 1.0)
        g = jnp.dot(x_ref[...], w_ref[...], preferred_element_type=jnp.float32)
        g_ref[...] = g
        m_ref[...] = g * dinv
        d_ref[...] = dinv

    return pl.pallas_call(
        body,
        grid=(N // NB,),
        in_specs=[
            pl.BlockSpec((NC, NB, 16), lambda i: (0, i, 0)),
            pl.BlockSpec((NB, DF), lambda i: (i, 0)),
            pl.BlockSpec((DF, H), lambda i: (0, 0)),
        ],
        out_specs=[
            pl.BlockSpec((NB, H), lambda i: (i, 0)),
            pl.BlockSpec((NB, H), lambda i: (i, 0)),
            pl.BlockSpec((NB, 1), lambda i: (i, 0)),
        ],
        out_shape=[
            jax.ShapeDtypeStruct((N, H), jnp.float32),
            jax.ShapeDtypeStruct((N, H), jnp.float32),
            jax.ShapeDtypeStruct((N, 1), jnp.float32),
        ],
    )(counts, x, W1)


def _tc_layer(acc, g, dinv, b, Wn):
    def body(a_ref, g_ref, d_ref, b_ref, w_ref, h_ref, gn_ref, mn_ref):
        d = d_ref[...]
        g = g_ref[...]
        agg = a_ref[0] + a_ref[1]
        h = jnp.tanh(d * agg + d * d * g + b_ref[...])
        h_ref[...] = h
        gn = jnp.dot(h, w_ref[...], preferred_element_type=jnp.float32)
        gn_ref[...] = gn
        mn_ref[...] = gn * d

    return pl.pallas_call(
        body,
        grid=(N // NB,),
        in_specs=[
            pl.BlockSpec((NC, NB, H), lambda i: (0, i, 0)),
            pl.BlockSpec((NB, H), lambda i: (i, 0)),
            pl.BlockSpec((NB, 1), lambda i: (i, 0)),
            pl.BlockSpec((1, H), lambda i: (0, 0)),
            pl.BlockSpec((H, H), lambda i: (0, 0)),
        ],
        out_specs=[
            pl.BlockSpec((NB, H), lambda i: (i, 0)),
            pl.BlockSpec((NB, H), lambda i: (i, 0)),
            pl.BlockSpec((NB, H), lambda i: (i, 0)),
        ],
        out_shape=[
            jax.ShapeDtypeStruct((N, H), jnp.float32),
            jax.ShapeDtypeStruct((N, H), jnp.float32),
            jax.ShapeDtypeStruct((N, H), jnp.float32),
        ],
    )(acc, g, dinv, b, Wn)


def _tc_layer3(acc, g, dinv, b, W4):
    def body(a_ref, g_ref, d_ref, b_ref, w_ref, h_ref, g4_ref, mn_ref):
        d = d_ref[...]
        g = g_ref[...]
        agg = a_ref[0] + a_ref[1]
        h = jnp.tanh(d * agg + d * d * g + b_ref[...])
        h_ref[...] = h
        g4 = jnp.dot(h, w_ref[...], preferred_element_type=jnp.float32)
        g4_ref[...] = g4
        mn_ref[:, 0:1] = g4 * d
        mn_ref[:, 1:16] = jnp.zeros((NB, 15), jnp.float32)

    return pl.pallas_call(
        body,
        grid=(N // NB,),
        in_specs=[
            pl.BlockSpec((NC, NB, H), lambda i: (0, i, 0)),
            pl.BlockSpec((NB, H), lambda i: (i, 0)),
            pl.BlockSpec((NB, 1), lambda i: (i, 0)),
            pl.BlockSpec((1, H), lambda i: (0, 0)),
            pl.BlockSpec((H, 1), lambda i: (0, 0)),
        ],
        out_specs=[
            pl.BlockSpec((NB, H), lambda i: (i, 0)),
            pl.BlockSpec((NB, 1), lambda i: (i, 0)),
            pl.BlockSpec((NB, 16), lambda i: (i, 0)),
        ],
        out_shape=[
            jax.ShapeDtypeStruct((N, H), jnp.float32),
            jax.ShapeDtypeStruct((N, 1), jnp.float32),
            jax.ShapeDtypeStruct((N, 16), jnp.float32),
        ],
    )(acc, g, dinv, b, W4)


def _tc_post4(acc4, h1, h2, h3, g4w, dinv, b4):
    def body(a_ref, h1_ref, h2_ref, h3_ref, g4_ref, d_ref, b4_ref,
             f_ref, kc_ref):
        d = d_ref[...]
        g4 = g4_ref[...]
        a0 = a_ref[0, :, 0:1] + a_ref[1, :, 0:1]
        h4 = jnp.tanh(d * a0 + d * d * g4 + b4_ref[...])
        f_ref[:, 0:H] = h1_ref[...]
        f_ref[:, H:2 * H] = h2_ref[...]
        f_ref[:, 2 * H:3 * H] = h3_ref[...]
        f_ref[:, 3 * H:3 * H + 1] = h4
        f_ref[:, 3 * H + 1:DP] = jnp.zeros((NB, DP - 3 * H - 1), jnp.float32)
        kc_ref[...] = h4

    return pl.pallas_call(
        body,
        grid=(N // NB,),
        in_specs=[
            pl.BlockSpec((NC, NB, 16), lambda i: (0, i, 0)),
            pl.BlockSpec((NB, H), lambda i: (i, 0)),
            pl.BlockSpec((NB, H), lambda i: (i, 0)),
            pl.BlockSpec((NB, H), lambda i: (i, 0)),
            pl.BlockSpec((NB, 1), lambda i: (i, 0)),
            pl.BlockSpec((NB, 1), lambda i: (i, 0)),
            pl.BlockSpec((1, 1), lambda i: (0, 0)),
        ],
        out_specs=[
            pl.BlockSpec((NB, DP), lambda i: (i, 0)),
            pl.BlockSpec((NB, 1), lambda i: (i, 0)),
        ],
        out_shape=[
            jax.ShapeDtypeStruct((NPAD, DP), jnp.float32),
            jax.ShapeDtypeStruct((N, 1), jnp.float32),
        ],
    )(acc4, h1, h2, h3, g4w, dinv, b4)


def _tc_rank(key_row, batch_row, key_col, batch_col):
    def body(kr_ref, br_ref, kc_ref, bc_ref, s_ref):
        i = pl.program_id(0)
        br_all = br_ref[...]
        b0 = bc_ref[0, 0]
        b1 = bc_ref[BI - 1, 0]
        lo = jnp.sum((br_all < b0).astype(jnp.int32))
        hi = jnp.sum((br_all <= b1).astype(jnp.int32))
        ki = kc_ref[...]
        bi = bc_ref[...]
        gi = i * BI + lax.broadcasted_iota(jnp.int32, (BI, 1), 0)

        def jbody(cc, cnt):
            off = pl.multiple_of(cc * CJ, CJ)
            kj = kr_ref[:, pl.ds(off, CJ)]
            bj = br_ref[:, pl.ds(off, CJ)]
            jj = off + lax.broadcasted_iota(jnp.int32, (1, CJ), 1)
            beat = (bj == bi) & ((kj > ki) | ((kj == ki) & (jj < gi)))
            return cnt + jnp.sum(beat.astype(jnp.int32), axis=1, keepdims=True)

        cnt = lax.fori_loop(lo // CJ, (hi + CJ - 1) // CJ, jbody,
                            jnp.zeros((BI, 1), jnp.int32))
        s_ref[...] = jnp.where((cnt < K) & (gi < N), bi * K + cnt, DUMP)

    return pl.pallas_call(
        body,
        grid=(NBLK,),
        in_specs=[
            pl.BlockSpec((1, NPAD), lambda i: (0, 0)),
            pl.BlockSpec((1, NPAD), lambda i: (0, 0)),
            pl.BlockSpec((BI, 1), lambda i: (i, 0)),
            pl.BlockSpec((BI, 1), lambda i: (i, 0)),
        ],
        out_specs=pl.BlockSpec((BI, 1), lambda i: (i, 0)),
        out_shape=jax.ShapeDtypeStruct((NPAD, 1), jnp.int32),
    )(key_row, batch_row, key_col, batch_col)


def _tc_head(pool, cw1r, cb1, w2r, cb2, lw1p, lb1, lw2, lb2):
    def body(p_ref, c1w, c1b, c2w, c2b, l1w, l1b, l2w, l2b, o_ref):
        P = (p_ref[0] + p_ref[1])[:, 0:3 * H + 1]
        c1 = jnp.dot(P, c1w[...], preferred_element_type=jnp.float32)
        c1 = jnp.maximum(c1 + c1b[...], 0.0)
        y = jnp.max(c1.reshape(GB, K // 2, 2, 16), axis=2)
        z = jnp.zeros((GB * 26, 32), jnp.float32) + c2b[...]
        for dt in range(5):
            ydt = y[:, dt:dt + 26, :].reshape(GB * 26, 16)
            z = z + jnp.dot(ydt, c2w[pl.ds(dt * 16, 16), :],
                            preferred_element_type=jnp.float32)
        z3 = jnp.maximum(z, 0.0).reshape(GB, 26, 32)
        q = l1b[...] + jnp.zeros((GB, DF), jnp.float32)
        for t in range(26):
            q = q + jnp.dot(z3[:, t, :], l1w[pl.ds(t * 32, 32), :],
                            preferred_element_type=jnp.float32)
        q = jnp.maximum(q, 0.0)
        o = jnp.dot(q, l2w[...], preferred_element_type=jnp.float32) + l2b[...]
        mo = jnp.max(o, axis=1, keepdims=True)
        e = jnp.exp(o - mo)
        o_ref[...] = e / jnp.sum(e, axis=1, keepdims=True)

    return pl.pallas_call(
        body,
        grid=(NG // GB,),
        in_specs=[
            pl.BlockSpec((NC, GB * K, DP), lambda i: (0, i, 0)),
            pl.BlockSpec((3 * H + 1, 16), lambda i: (0, 0)),
            pl.BlockSpec((1, 16), lambda i: (0, 0)),
            pl.BlockSpec((80, 32), lambda i: (0, 0)),
            pl.BlockSpec((1, 32), lambda i: (0, 0)),
            pl.BlockSpec((832, DF), lambda i: (0, 0)),
            pl.BlockSpec((1, DF), lambda i: (0, 0)),
            pl.BlockSpec((DF, NCLS), lambda i: (0, 0)),
            pl.BlockSpec((1, NCLS), lambda i: (0, 0)),
        ],
        out_specs=pl.BlockSpec((GB, NCLS), lambda i: (i, 0)),
        out_shape=jax.ShapeDtypeStruct((NG, NCLS), jnp.float32),
    )(pool, cw1r, cb1, w2r, cb2, lw1p, lb1, lw2, lb2)


def kernel(x, edge_index, batch, W1, b1, W2, b2, W3, b3, W4, b4,
           cw1, cb1, cw2, cb2, lw1, lb1, lw2, lb2):
    src = edge_index[0].astype(jnp.int32)
    dst = edge_index[1].astype(jnp.int32)
    srcw = jnp.concatenate(
        [src, jnp.zeros((EPAD - E,), jnp.int32)]).reshape(NW, ENCH, ECH)
    dstw = jnp.concatenate(
        [dst, jnp.full((EPAD - E,), NACC - 1, jnp.int32)]).reshape(NW, ENCH, ECH)

    counts = _sc_counts(dstw)

    g1, m1, dinv = _tc_prep1(x, W1, counts)

    acc1 = _sc_agg(m1, srcw, dstw, H)
    h1, g2, m2 = _tc_layer(acc1, g1, dinv, b1.reshape(1, H), W2)
    acc2 = _sc_agg(m2, srcw, dstw, H)
    h2, g3, m3 = _tc_layer(acc2, g2, dinv, b2.reshape(1, H), W3)
    acc3 = _sc_agg(m3, srcw, dstw, H)
    h3, g4w, m4n = _tc_layer3(acc3, g3, dinv, b3.reshape(1, H), W4)
    acc4 = _sc_agg(m4n, srcw, dstw, 16)
    featp, key_col = _tc_post4(acc4, h1, h2, h3, g4w, dinv, b4.reshape(1, 1))

    key_row = jnp.pad(key_col.reshape(1, N), ((0, 0), (0, NPAD - N)))
    batch32 = batch.astype(jnp.int32)
    batch_pad = jnp.concatenate([batch32, jnp.full((NPAD - N,), NG, jnp.int32)])
    key_colp = jnp.pad(key_col, ((0, NPAD - N), (0, 0)))
    batch_row = batch_pad.reshape(1, NPAD)
    batch_col = batch_pad.reshape(NPAD, 1)
    slot = _tc_rank(key_row, batch_row, key_colp, batch_col)

    slotw = slot.reshape(NW, NPW // PCH, PCH)
    pool = _sc_pool(featp, slotw)

    cw1r = cw1[:, 0, :].T
    w2r = cw2.transpose(2, 1, 0).reshape(80, 32)
    lw1p = lw1.reshape(32, 26, DF).transpose(1, 0, 2).reshape(832, DF)
    return _tc_head(pool, cw1r, cb1.reshape(1, 16), w2r, cb2.reshape(1, 32),
                    lw1p, lb1.reshape(1, DF), lw2, lb2.reshape(1, NCLS))


# speedup vs baseline: 1.7910x; 1.0127x over previous
import functools

import jax
import jax.numpy as jnp
from jax import lax
from jax.experimental import pallas as pl
from jax.experimental.pallas import tpu as pltpu
from jax.experimental.pallas import tpu_sc as plsc

N = 10000
E = 320000
DF = 128
H = 32
NG = 256
K = 60
NCLS = 4
DP = 128

NC, NS = 2, 16
NW = NC * NS
ECH = 80
EPW = 10240
ENCH = EPW // ECH
EPAD = NW * EPW
PCH = 80
NACC = 10112
NPR = NACC // NS
NPW = 320
NPAD = NW * NPW
SLOTS = NG * K
DUMP = SLOTS
SLOTS_PAD = 16000
NB = 5000
BI = 256
NBLK = 40
CJ = 512
GB = 64

_mesh = plsc.VectorSubcoreMesh(core_axis_name="c", subcore_axis_name="s")
_sc_params = pltpu.CompilerParams(use_tc_tiling_on_sc=False)


def _sc_counts(dstw):
    @functools.partial(
        pl.kernel, mesh=_mesh, compiler_params=_sc_params,
        out_type=jax.ShapeDtypeStruct((NC, NACC, 16), jnp.float32),
        scratch_types=[
            pltpu.VMEM((ENCH, ECH), jnp.int32),
            pltpu.VMEM((ECH, 16), jnp.float32),
            pltpu.VMEM((NPR, 16), jnp.float32),
            pltpu.VMEM_SHARED((NACC, 16), jnp.float32),
        ])
    def k(dst_hbm, out_hbm, dst_v, ones_v, zero_v, acc_sh):
        c = lax.axis_index("c")
        s = lax.axis_index("s")
        wid = c * NS + s
        pltpu.sync_copy(dst_hbm.at[wid], dst_v)
        z16 = jnp.zeros((16,), jnp.float32)
        o16 = jnp.full((16,), 1.0, jnp.float32)

        @pl.loop(0, ECH)
        def _(i):
            ones_v[i, pl.ds(0, 16)] = o16

        @pl.loop(0, NPR)
        def _(i):
            zero_v[i, pl.ds(0, 16)] = z16

        pltpu.sync_copy(zero_v, acc_sh.at[pl.ds(s * NPR, NPR)])
        plsc.subcore_barrier()

        @pl.loop(0, ENCH)
        def _(j):
            pltpu.sync_copy(ones_v, acc_sh.at[dst_v.at[j]], add=True)

        plsc.subcore_barrier()
        pltpu.sync_copy(acc_sh.at[pl.ds(s * NPR, NPR)],
                        out_hbm.at[c, pl.ds(s * NPR, NPR)])

    return k(dstw)


def _sc_agg(m, srcw, dstw, w):
    @functools.partial(
        pl.kernel, mesh=_mesh, compiler_params=_sc_params,
        out_type=jax.ShapeDtypeStruct((NC, NACC, w), jnp.float32),
        scratch_types=[
            pltpu.VMEM((ENCH, ECH), jnp.int32),
            pltpu.VMEM((ENCH, ECH), jnp.int32),
            pltpu.VMEM((ECH, w), jnp.float32),
            pltpu.VMEM((ECH, w), jnp.float32),
            pltpu.VMEM((ECH, w), jnp.float32),
            pltpu.VMEM((ECH, w), jnp.float32),
            pltpu.VMEM((NPR, w), jnp.float32),
            pltpu.VMEM_SHARED((NACC, w), jnp.float32),
            pltpu.VMEM_SHARED((N, w), jnp.float32),
            pltpu.SemaphoreType.DMA,
            pltpu.SemaphoreType.DMA,
            pltpu.SemaphoreType.DMA,
            pltpu.SemaphoreType.DMA,
        ])
    def k(m_hbm, src_hbm, dst_hbm, out_hbm, src_v, dst_v, r0, r1, r2, r3,
          zero_v, acc_sh, m_sh, s0, s1, s2, s3):
        c = lax.axis_index("c")
        s = lax.axis_index("s")
        wid = c * NS + s
        pltpu.sync_copy(src_hbm.at[wid], src_v)
        pltpu.sync_copy(dst_hbm.at[wid], dst_v)
        pltpu.sync_copy(m_hbm.at[pl.ds(s * (N // NS), N // NS)],
                        m_sh.at[pl.ds(s * (N // NS), N // NS)])
        z16 = jnp.zeros((16,), jnp.float32)

        def gath(j, rbuf, sem):
            pltpu.async_copy(m_sh.at[src_v.at[j]], rbuf, sem)

        def gwait(rbuf, sem):
            pltpu.make_async_copy(m_sh.at[src_v.at[0]], rbuf, sem).wait()

        @pl.loop(0, NPR)
        def _(i):
            @pl.loop(0, w, step=16)
            def _(jc):
                zero_v[i, pl.ds(jc, 16)] = z16

        pltpu.sync_copy(zero_v, acc_sh.at[pl.ds(s * NPR, NPR)])
        plsc.subcore_barrier()
        gath(0, r0, s0)
        gath(1, r1, s1)
        gath(2, r2, s2)

        @pl.loop(0, ENCH, step=4)
        def _(j):
            gath(j + 3, r3, s3)
            gwait(r0, s0)
            pltpu.sync_copy(r0, acc_sh.at[dst_v.at[j]], add=True)

            @pl.when(j + 4 < ENCH)
            def _():
                gath(j + 4, r0, s0)

            gwait(r1, s1)
            pltpu.sync_copy(r1, acc_sh.at[dst_v.at[j + 1]], add=True)

            @pl.when(j + 5 < ENCH)
            def _():
                gath(j + 5, r1, s1)

            gwait(r2, s2)
            pltpu.sync_copy(r2, acc_sh.at[dst_v.at[j + 2]], add=True)

            @pl.when(j + 6 < ENCH)
            def _():
                gath(j + 6, r2, s2)

            gwait(r3, s3)
            pltpu.sync_copy(r3, acc_sh.at[dst_v.at[j + 3]], add=True)

            @pl.when(j + 7 < ENCH)
            def _():
                gath(j + 7, r3, s3)

        plsc.subcore_barrier()
        pltpu.sync_copy(acc_sh.at[pl.ds(s * NPR, NPR)],
                        out_hbm.at[c, pl.ds(s * NPR, NPR)])

    return k(m, srcw, dstw)


def _sc_pool(featp, slotw):
    @functools.partial(
        pl.kernel, mesh=_mesh, compiler_params=_sc_params,
        out_type=jax.ShapeDtypeStruct((NC, SLOTS_PAD, DP), jnp.float32),
        scratch_types=[
            pltpu.VMEM((NPW // PCH, PCH), jnp.int32),
            pltpu.VMEM((NPW, DP), jnp.float32),
            pltpu.VMEM((200, DP), jnp.float32),
        ])
    def k(f_hbm, sl_hbm, out_hbm, sl_v, f_v, zero_v):
        c = lax.axis_index("c")
        s = lax.axis_index("s")
        wid = c * NS + s
        z16 = jnp.zeros((16,), jnp.float32)

        @pl.loop(0, 200)
        def _(i):
            @pl.loop(0, DP, step=16)
            def _(j):
                zero_v[i, pl.ds(j, 16)] = z16

        @pl.loop(0, SLOTS_PAD // NS // 200)
        def _(t):
            pltpu.sync_copy(
                zero_v,
                out_hbm.at[c, pl.ds(s * (SLOTS_PAD // NS) + t * 200, 200)])

        pltpu.sync_copy(f_hbm.at[pl.ds(wid * NPW, NPW)], f_v)
        pltpu.sync_copy(sl_hbm.at[wid], sl_v)
        plsc.subcore_barrier()

        @pl.loop(0, NPW // PCH)
        def _(t):
            pltpu.sync_copy(f_v.at[pl.ds(t * PCH, PCH)],
                            out_hbm.at[c].at[sl_v.at[t]])

    return k(featp, slotw)


def _tc_prep1(x, W1, counts):
    def body(c_ref, x_ref, w_ref, g_ref, m_ref, d_ref):
        cnt = c_ref[0, :, 0:1] + c_ref[1, :, 0:1]
        dinv = lax.rsqrt(cnt + 1.0)
        g = jnp.dot(x_ref[...], w_ref[...], preferred_element_type=jnp.float32)
        g_ref[...] = g
        m_ref[...] = g * dinv
        d_ref[...] = dinv

    return pl.pallas_call(
        body,
        grid=(N // NB,),
        in_specs=[
            pl.BlockSpec((NC, NB, 16), lambda i: (0, i, 0)),
            pl.BlockSpec((NB, DF), lambda i: (i, 0)),
            pl.BlockSpec((DF, H), lambda i: (0, 0)),
        ],
        out_specs=[
            pl.BlockSpec((NB, H), lambda i: (i, 0)),
            pl.BlockSpec((NB, H), lambda i: (i, 0)),
            pl.BlockSpec((NB, 1), lambda i: (i, 0)),
        ],
        out_shape=[
            jax.ShapeDtypeStruct((N, H), jnp.float32),
            jax.ShapeDtypeStruct((N, H), jnp.float32),
            jax.ShapeDtypeStruct((N, 1), jnp.float32),
        ],
    )(counts, x, W1)


def _tc_layer(acc, g, dinv, b, Wn):
    def body(a_ref, g_ref, d_ref, b_ref, w_ref, h_ref, gn_ref, mn_ref):
        d = d_ref[...]
        g = g_ref[...]
        agg = a_ref[0] + a_ref[1]
        h = jnp.tanh(d * agg + d * d * g + b_ref[...])
        h_ref[...] = h
        gn = jnp.dot(h, w_ref[...], preferred_element_type=jnp.float32)
        gn_ref[...] = gn
        mn_ref[...] = gn * d

    return pl.pallas_call(
        body,
        grid=(N // NB,),
        in_specs=[
            pl.BlockSpec((NC, NB, H), lambda i: (0, i, 0)),
            pl.BlockSpec((NB, H), lambda i: (i, 0)),
            pl.BlockSpec((NB, 1), lambda i: (i, 0)),
            pl.BlockSpec((1, H), lambda i: (0, 0)),
            pl.BlockSpec((H, H), lambda i: (0, 0)),
        ],
        out_specs=[
            pl.BlockSpec((NB, H), lambda i: (i, 0)),
            pl.BlockSpec((NB, H), lambda i: (i, 0)),
            pl.BlockSpec((NB, H), lambda i: (i, 0)),
        ],
        out_shape=[
            jax.ShapeDtypeStruct((N, H), jnp.float32),
            jax.ShapeDtypeStruct((N, H), jnp.float32),
            jax.ShapeDtypeStruct((N, H), jnp.float32),
        ],
    )(acc, g, dinv, b, Wn)


def _tc_layer3(acc, g, dinv, b, W4):
    def body(a_ref, g_ref, d_ref, b_ref, w_ref, h_ref, g4_ref, mn_ref):
        d = d_ref[...]
        g = g_ref[...]
        agg = a_ref[0] + a_ref[1]
        h = jnp.tanh(d * agg + d * d * g + b_ref[...])
        h_ref[...] = h
        g4 = jnp.dot(h, w_ref[...], preferred_element_type=jnp.float32)
        g4_ref[...] = g4
        mn_ref[:, 0:1] = g4 * d
        mn_ref[:, 1:16] = jnp.zeros((NB, 15), jnp.float32)

    return pl.pallas_call(
        body,
        grid=(N // NB,),
        in_specs=[
            pl.BlockSpec((NC, NB, H), lambda i: (0, i, 0)),
            pl.BlockSpec((NB, H), lambda i: (i, 0)),
            pl.BlockSpec((NB, 1), lambda i: (i, 0)),
            pl.BlockSpec((1, H), lambda i: (0, 0)),
            pl.BlockSpec((H, 1), lambda i: (0, 0)),
        ],
        out_specs=[
            pl.BlockSpec((NB, H), lambda i: (i, 0)),
            pl.BlockSpec((NB, 1), lambda i: (i, 0)),
            pl.BlockSpec((NB, 16), lambda i: (i, 0)),
        ],
        out_shape=[
            jax.ShapeDtypeStruct((N, H), jnp.float32),
            jax.ShapeDtypeStruct((N, 1), jnp.float32),
            jax.ShapeDtypeStruct((N, 16), jnp.float32),
        ],
    )(acc, g, dinv, b, W4)


def _tc_post4(acc4, h1, h2, h3, g4w, dinv, b4):
    def body(a_ref, h1_ref, h2_ref, h3_ref, g4_ref, d_ref, b4_ref,
             f_ref, kc_ref):
        d = d_ref[...]
        g4 = g4_ref[...]
        a0 = a_ref[0, :, 0:1] + a_ref[1, :, 0:1]
        h4 = jnp.tanh(d * a0 + d * d * g4 + b4_ref[...])
        f_ref[:, 0:H] = h1_ref[...]
        f_ref[:, H:2 * H] = h2_ref[...]
        f_ref[:, 2 * H:3 * H] = h3_ref[...]
        f_ref[:, 3 * H:3 * H + 1] = h4
        f_ref[:, 3 * H + 1:DP] = jnp.zeros((NB, DP - 3 * H - 1), jnp.float32)
        kc_ref[...] = h4

    return pl.pallas_call(
        body,
        grid=(N // NB,),
        in_specs=[
            pl.BlockSpec((NC, NB, 16), lambda i: (0, i, 0)),
            pl.BlockSpec((NB, H), lambda i: (i, 0)),
            pl.BlockSpec((NB, H), lambda i: (i, 0)),
            pl.BlockSpec((NB, H), lambda i: (i, 0)),
            pl.BlockSpec((NB, 1), lambda i: (i, 0)),
            pl.BlockSpec((NB, 1), lambda i: (i, 0)),
            pl.BlockSpec((1, 1), lambda i: (0, 0)),
        ],
        out_specs=[
            pl.BlockSpec((NB, DP), lambda i: (i, 0)),
            pl.BlockSpec((NB, 1), lambda i: (i, 0)),
        ],
        out_shape=[
            jax.ShapeDtypeStruct((NPAD, DP), jnp.float32),
            jax.ShapeDtypeStruct((N, 1), jnp.float32),
        ],
    )(acc4, h1, h2, h3, g4w, dinv, b4)


def _tc_bounds(batch_row, b0c, b1c):
    def body(br_ref, b0_ref, b1_ref, lo_ref, hi_ref):
        br = br_ref[...]
        lo_ref[...] = jnp.sum((br < b0_ref[...]).astype(jnp.int32),
                              axis=1, keepdims=True)
        hi_ref[...] = jnp.sum((br <= b1_ref[...]).astype(jnp.int32),
                              axis=1, keepdims=True)

    return pl.pallas_call(
        body,
        grid=(1,),
        in_specs=[
            pl.BlockSpec((1, NPAD), lambda i: (0, 0)),
            pl.BlockSpec((NBLK, 1), lambda i: (0, 0)),
            pl.BlockSpec((NBLK, 1), lambda i: (0, 0)),
        ],
        out_specs=[
            pl.BlockSpec((NBLK, 1), lambda i: (0, 0)),
            pl.BlockSpec((NBLK, 1), lambda i: (0, 0)),
        ],
        out_shape=[
            jax.ShapeDtypeStruct((NBLK, 1), jnp.int32),
            jax.ShapeDtypeStruct((NBLK, 1), jnp.int32),
        ],
    )(batch_row, b0c, b1c)


def _tc_rank(key_row, batch_row, key_col, batch_col, jlo, jhi):
    def body(kr_ref, br_ref, kc_ref, bc_ref, lo_ref, hi_ref, s_ref):
        i = pl.program_id(0)
        lo = lo_ref[i, 0]
        hi = hi_ref[i, 0]
        ki = kc_ref[...]
        bi = bc_ref[...]
        gi = i * BI + lax.broadcasted_iota(jnp.int32, (BI, 1), 0)

        def jbody(cc, cnt):
            off = pl.multiple_of(cc * CJ, CJ)
            kj = kr_ref[:, pl.ds(off, CJ)]
            bj = br_ref[:, pl.ds(off, CJ)]
            jj = off + lax.broadcasted_iota(jnp.int32, (1, CJ), 1)
            beat = (bj == bi) & ((kj > ki) | ((kj == ki) & (jj < gi)))
            return cnt + jnp.sum(beat.astype(jnp.int32), axis=1, keepdims=True)

        cnt = lax.fori_loop(lo // CJ, (hi + CJ - 1) // CJ, jbody,
                            jnp.zeros((BI, 1), jnp.int32))
        s_ref[...] = jnp.where((cnt < K) & (gi < N), bi * K + cnt, DUMP)

    return pl.pallas_call(
        body,
        grid=(NBLK,),
        in_specs=[
            pl.BlockSpec((1, NPAD), lambda i: (0, 0)),
            pl.BlockSpec((1, NPAD), lambda i: (0, 0)),
            pl.BlockSpec((BI, 1), lambda i: (i, 0)),
            pl.BlockSpec((BI, 1), lambda i: (i, 0)),
            pl.BlockSpec(memory_space=pltpu.SMEM),
            pl.BlockSpec(memory_space=pltpu.SMEM),
        ],
        out_specs=pl.BlockSpec((BI, 1), lambda i: (i, 0)),
        out_shape=jax.ShapeDtypeStruct((NPAD, 1), jnp.int32),
    )(key_row, batch_row, key_col, batch_col, jlo, jhi)


def _tc_head(pool, cw1r, cb1, w2r, cb2, lw1p, lb1, lw2, lb2):
    def body(p_ref, c1w, c1b, c2w, c2b, l1w, l1b, l2w, l2b, o_ref):
        P = (p_ref[0] + p_ref[1])[:, 0:3 * H + 1]
        c1 = jnp.dot(P, c1w[...], preferred_element_type=jnp.float32)
        c1 = jnp.maximum(c1 + c1b[...], 0.0)
        y = jnp.max(c1.reshape(GB, K // 2, 2, 16), axis=2)
        z = jnp.zeros((GB * 26, 32), jnp.float32) + c2b[...]
        for dt in range(5):
            ydt = y[:, dt:dt + 26, :].reshape(GB * 26, 16)
            z = z + jnp.dot(ydt, c2w[pl.ds(dt * 16, 16), :],
                            preferred_element_type=jnp.float32)
        z3 = jnp.maximum(z, 0.0).reshape(GB, 26, 32)
        q = l1b[...] + jnp.zeros((GB, DF), jnp.float32)
        for t in range(26):
            q = q + jnp.dot(z3[:, t, :], l1w[pl.ds(t * 32, 32), :],
                            preferred_element_type=jnp.float32)
        q = jnp.maximum(q, 0.0)
        o = jnp.dot(q, l2w[...], preferred_element_type=jnp.float32) + l2b[...]
        mo = jnp.max(o, axis=1, keepdims=True)
        e = jnp.exp(o - mo)
        o_ref[...] = e / jnp.sum(e, axis=1, keepdims=True)

    return pl.pallas_call(
        body,
        grid=(NG // GB,),
        in_specs=[
            pl.BlockSpec((NC, GB * K, DP), lambda i: (0, i, 0)),
            pl.BlockSpec((3 * H + 1, 16), lambda i: (0, 0)),
            pl.BlockSpec((1, 16), lambda i: (0, 0)),
            pl.BlockSpec((80, 32), lambda i: (0, 0)),
            pl.BlockSpec((1, 32), lambda i: (0, 0)),
            pl.BlockSpec((832, DF), lambda i: (0, 0)),
            pl.BlockSpec((1, DF), lambda i: (0, 0)),
            pl.BlockSpec((DF, NCLS), lambda i: (0, 0)),
            pl.BlockSpec((1, NCLS), lambda i: (0, 0)),
        ],
        out_specs=pl.BlockSpec((GB, NCLS), lambda i: (i, 0)),
        out_shape=jax.ShapeDtypeStruct((NG, NCLS), jnp.float32),
    )(pool, cw1r, cb1, w2r, cb2, lw1p, lb1, lw2, lb2)


def kernel(x, edge_index, batch, W1, b1, W2, b2, W3, b3, W4, b4,
           cw1, cb1, cw2, cb2, lw1, lb1, lw2, lb2):
    src = edge_index[0].astype(jnp.int32)
    dst = edge_index[1].astype(jnp.int32)
    srcw = jnp.concatenate(
        [src, jnp.zeros((EPAD - E,), jnp.int32)]).reshape(NW, ENCH, ECH)
    dstw = jnp.concatenate(
        [dst, jnp.full((EPAD - E,), NACC - 1, jnp.int32)]).reshape(NW, ENCH, ECH)

    counts = _sc_counts(dstw)

    g1, m1, dinv = _tc_prep1(x, W1, counts)

    acc1 = _sc_agg(m1, srcw, dstw, H)
    h1, g2, m2 = _tc_layer(acc1, g1, dinv, b1.reshape(1, H), W2)
    acc2 = _sc_agg(m2, srcw, dstw, H)
    h2, g3, m3 = _tc_layer(acc2, g2, dinv, b2.reshape(1, H), W3)
    acc3 = _sc_agg(m3, srcw, dstw, H)
    h3, g4w, m4n = _tc_layer3(acc3, g3, dinv, b3.reshape(1, H), W4)
    acc4 = _sc_agg(m4n, srcw, dstw, 16)
    featp, key_col = _tc_post4(acc4, h1, h2, h3, g4w, dinv, b4.reshape(1, 1))

    key_row = jnp.pad(key_col.reshape(1, N), ((0, 0), (0, NPAD - N)))
    batch32 = batch.astype(jnp.int32)
    batch_pad = jnp.concatenate([batch32, jnp.full((NPAD - N,), NG, jnp.int32)])
    key_colp = jnp.pad(key_col, ((0, NPAD - N), (0, 0)))
    batch_row = batch_pad.reshape(1, NPAD)
    batch_col = batch_pad.reshape(NPAD, 1)
    b0c = batch_pad[0::BI].reshape(NBLK, 1)
    b1c = batch_pad[BI - 1::BI].reshape(NBLK, 1)
    jlo, jhi = _tc_bounds(batch_row, b0c, b1c)
    slot = _tc_rank(key_row, batch_row, key_colp, batch_col, jlo, jhi)

    slotw = slot.reshape(NW, NPW // PCH, PCH)
    pool = _sc_pool(featp, slotw)

    cw1r = cw1[:, 0, :].T
    w2r = cw2.transpose(2, 1, 0).reshape(80, 32)
    lw1p = lw1.reshape(32, 26, DF).transpose(1, 0, 2).reshape(832, DF)
    return _tc_head(pool, cw1r, cb1.reshape(1, 16), w2r, cb2.reshape(1, 32),
                    lw1p, lb1.reshape(1, DF), lw2, lb2.reshape(1, NCLS))
